# Initial kernel scaffold; baseline (speedup 1.0000x reference)
import jax
import jax.numpy as jnp
from jax.experimental import pallas as pl


def kernel(x_sub, x_hru_agr, x_hru_urb, ei_ss, ei_as, ei_us, ei_sa, ei_su, params):
    raise NotImplementedError("write your pallas kernel here")


# speedup vs baseline: 6.6877x; 6.6877x over previous
import functools

import jax
import jax.numpy as jnp
from jax import lax
from jax.experimental import pallas as pl
from jax.experimental.pallas import tpu as pltpu
from jax.experimental.pallas import tpu_sc as plsc

N = 10000
NP = 10240
D = 128
OUT = 16
NC = 2
NS = 16
NW = NC * NS
CH = 100
E_SS_ = 320000
E_AS_ = 400000
STRIPE = NP // NS

def _seg_accumulate(table, srcv, dstv, rows, acc, nchunks):
  def body(j, _):
    pltpu.sync_copy(table.at[srcv.at[j]], rows)
    pltpu.sync_copy(rows, acc.at[dstv.at[j]], add=True)
    return ()
  lax.fori_loop(0, nchunks, body, (), unroll=False)


def _stage_indices(src_hbm, dst_hbm, wid, srcv, dstv, nchunks):
  pltpu.sync_copy(src_hbm.at[wid], srcv.at[pl.ds(0, nchunks)])
  pltpu.sync_copy(dst_hbm.at[wid], dstv.at[pl.ds(0, nchunks)])


def _zero_acc(zrows, acc, s):
  pltpu.sync_copy(zrows, acc.at[pl.ds(s * STRIPE, STRIPE)])


def _copy_out(acc, out, c, s):
  pltpu.sync_copy(acc.at[pl.ds(s * STRIPE, STRIPE)],
                  out.at[c, pl.ds(s * STRIPE, STRIPE)])


def _sc_multi_body(xs, xa, xu, src_as, dst_as, src_us, dst_us, src_ss, dst_ss,
                   zrows, zvec, ones_h,
                   p_as, p_us, p_ss, p_cnt,
                   acc, cntacc, srcv, dstv, rows, ones):
  c = lax.axis_index("c")
  s = lax.axis_index("s")
  wid = c * NS + s
  n_asus = E_AS_ // NW // CH
  n_ss = E_SS_ // NW // CH

  pltpu.sync_copy(ones_h, ones)

  _zero_acc(zrows, acc, s)
  plsc.subcore_barrier()
  _stage_indices(src_as, dst_as, wid, srcv, dstv, n_asus)
  _seg_accumulate(xa, srcv, dstv, rows, acc, n_asus)
  plsc.subcore_barrier()
  _copy_out(acc, p_as, c, s)
  plsc.subcore_barrier()

  _zero_acc(zrows, acc, s)
  plsc.subcore_barrier()
  _stage_indices(src_us, dst_us, wid, srcv, dstv, n_asus)
  _seg_accumulate(xu, srcv, dstv, rows, acc, n_asus)
  plsc.subcore_barrier()
  _copy_out(acc, p_us, c, s)
  plsc.subcore_barrier()

  _zero_acc(zrows, acc, s)
  pltpu.sync_copy(zvec, cntacc.at[pl.ds(s * STRIPE, STRIPE)])
  plsc.subcore_barrier()
  _stage_indices(src_ss, dst_ss, wid, srcv, dstv, n_ss)

  def body(j, _):
    pltpu.sync_copy(xs.at[srcv.at[j]], rows)
    pltpu.sync_copy(rows, acc.at[dstv.at[j]], add=True)
    pltpu.sync_copy(ones, cntacc.at[dstv.at[j]], add=True)
    return ()
  lax.fori_loop(0, n_ss, body, (), unroll=False)

  plsc.subcore_barrier()
  _copy_out(acc, p_ss, c, s)
  pltpu.sync_copy(cntacc.at[pl.ds(s * STRIPE, STRIPE)],
                  p_cnt.at[c, pl.ds(s * STRIPE, STRIPE)])


def _sc_ss_body(xs, src_ss, dst_ss, zrows, p_ss, acc, srcv, dstv, rows):
  c = lax.axis_index("c")
  s = lax.axis_index("s")
  wid = c * NS + s
  n_ss = E_SS_ // NW // CH
  _zero_acc(zrows, acc, s)
  plsc.subcore_barrier()
  _stage_indices(src_ss, dst_ss, wid, srcv, dstv, n_ss)
  _seg_accumulate(xs, srcv, dstv, rows, acc, n_ss)
  plsc.subcore_barrier()
  _copy_out(acc, p_ss, c, s)


@functools.cache
def _sc_kernels():
  mesh = plsc.VectorSubcoreMesh(
      core_axis_name="c", subcore_axis_name="s",
      num_cores=NC, num_subcores=NS)
  f32 = jnp.float32
  sc_multi = pl.kernel(
      _sc_multi_body,
      out_type=(
          jax.ShapeDtypeStruct((NC, NP, D), f32),
          jax.ShapeDtypeStruct((NC, NP, D), f32),
          jax.ShapeDtypeStruct((NC, NP, D), f32),
          jax.ShapeDtypeStruct((NC, NP), f32),
      ),
      mesh=mesh,
      scratch_types=[
          pltpu.VMEM_SHARED((NP, D), f32),
          pltpu.VMEM_SHARED((NP,), f32),
          pltpu.VMEM((E_AS_ // NW // CH, CH), jnp.int32),
          pltpu.VMEM((E_AS_ // NW // CH, CH), jnp.int32),
          pltpu.VMEM((CH, D), f32),
          pltpu.VMEM((CH,), f32),
      ],
  )
  sc_ss = pl.kernel(
      _sc_ss_body,
      out_type=jax.ShapeDtypeStruct((NC, NP, D), f32),
      mesh=mesh,
      scratch_types=[
          pltpu.VMEM_SHARED((NP, D), f32),
          pltpu.VMEM((E_SS_ // NW // CH, CH), jnp.int32),
          pltpu.VMEM((E_SS_ // NW // CH, CH), jnp.int32),
          pltpu.VMEM((CH, D), f32),
      ],
  )
  return sc_multi, sc_ss


BR = 1024


def _tc_prep_body(pas_ref, pus_ref, wa_ref, wu_ref, b_ref, out_ref):
  a = pas_ref[0] + pas_ref[1]
  u = pus_ref[0] + pus_ref[1]
  o = jnp.dot(a, wa_ref[0], preferred_element_type=jnp.float32)
  o = o + jnp.dot(u, wu_ref[0], preferred_element_type=jnp.float32)
  out_ref[0] = o + b_ref[0, 0][None, :]


def _tc_prep(p_as, p_us, wa, wu, b):
  return pl.pallas_call(
      _tc_prep_body,
      out_shape=jax.ShapeDtypeStruct((3, NP, D), jnp.float32),
      grid=(3, NP // BR),
      in_specs=[
          pl.BlockSpec((NC, BR, D), lambda l, i: (0, i, 0)),
          pl.BlockSpec((NC, BR, D), lambda l, i: (0, i, 0)),
          pl.BlockSpec((1, D, D), lambda l, i: (l, 0, 0)),
          pl.BlockSpec((1, D, D), lambda l, i: (l, 0, 0)),
          pl.BlockSpec((1, 1, D), lambda l, i: (l, 0, 0)),
      ],
      out_specs=pl.BlockSpec((1, BR, D), lambda l, i: (l, i, 0)),
  )(p_as, p_us, wa, wu, b)


def _tc_layer_body(mean, p_ref, cnt_ref, sub_ref, wl_ref, wr_ref, c_ref,
                   out_ref):
  y = p_ref[0] + p_ref[1]
  if mean:
    cnt = cnt_ref[0] + cnt_ref[1]
    y = y * (1.0 / jnp.maximum(cnt, 1.0))[:, None]
  o = jnp.dot(y, wl_ref[...], preferred_element_type=jnp.float32)
  o = o + jnp.dot(sub_ref[...], wr_ref[...], preferred_element_type=jnp.float32)
  out_ref[...] = jnp.maximum(o + c_ref[...], 0.0)


def _tc_layer(mean, p, cnt, sub, wl, wr, cterm):
  return pl.pallas_call(
      functools.partial(_tc_layer_body, mean),
      out_shape=jax.ShapeDtypeStruct((NP, D), jnp.float32),
      grid=(NP // BR,),
      in_specs=[
          pl.BlockSpec((NC, BR, D), lambda i: (0, i, 0)),
          pl.BlockSpec((NC, BR), lambda i: (0, i)),
          pl.BlockSpec((BR, D), lambda i: (i, 0)),
          pl.BlockSpec((D, D), lambda i: (0, 0)),
          pl.BlockSpec((D, D), lambda i: (0, 0)),
          pl.BlockSpec((BR, D), lambda i: (i, 0)),
      ],
      out_specs=pl.BlockSpec((BR, D), lambda i: (i, 0)),
  )(p, cnt, sub, wl, wr, cterm)


def _tc_final_body(sub_ref, wf_ref, bf_ref, out_ref):
  logits = jnp.dot(sub_ref[...], wf_ref[...],
                   preferred_element_type=jnp.float32) + bf_ref[0][None, :]
  m = jnp.max(logits, axis=1, keepdims=True)
  e = jnp.exp(logits - m)
  out_ref[...] = e / jnp.sum(e, axis=1, keepdims=True)


def _tc_final(sub, wf, bf):
  return pl.pallas_call(
      _tc_final_body,
      out_shape=jax.ShapeDtypeStruct((NP, D), jnp.float32),
      grid=(NP // BR,),
      in_specs=[
          pl.BlockSpec((BR, D), lambda i: (i, 0)),
          pl.BlockSpec((D, D), lambda i: (0, 0)),
          pl.BlockSpec((1, D), lambda i: (0, 0)),
      ],
      out_specs=pl.BlockSpec((BR, D), lambda i: (i, 0)),
  )(sub, wf, bf)


def kernel(x_sub, x_hru_agr, x_hru_urb, ei_ss, ei_as, ei_us, ei_sa, ei_su,
           params):
  del ei_sa, ei_su
  f32 = jnp.float32
  xs = jnp.zeros((NP, D), f32).at[:N].set(x_sub.astype(f32))

  def reshape_ei(ei):
    src = ei[0].astype(jnp.int32).reshape(NW, -1, CH)
    dst = ei[1].astype(jnp.int32).reshape(NW, -1, CH)
    return src, dst

  src_ss, dst_ss = reshape_ei(ei_ss)
  src_as, dst_as = reshape_ei(ei_as)
  src_us, dst_us = reshape_ei(ei_us)

  zrows = jnp.zeros((STRIPE, D), f32)
  zvec = jnp.zeros((STRIPE,), f32)
  ones_h = jnp.ones((CH,), f32)

  sc_multi, sc_ss = _sc_kernels()
  p_as, p_us, p_ss0, p_cnt = sc_multi(
      xs, x_hru_agr.astype(f32), x_hru_urb.astype(f32),
      src_as, dst_as, src_us, dst_us, src_ss, dst_ss, zrows, zvec, ones_h)

  wa = jnp.stack([params[f"Wl_as_{l}"] for l in range(3)])
  wu = jnp.stack([params[f"Wl_us_{l}"] for l in range(3)])
  b = jnp.stack([params[f"bl_ss_{l}"] + params[f"bl_as_{l}"]
                 + params[f"bl_us_{l}"] for l in range(3)])[:, None, :]
  c_all = _tc_prep(p_as, p_us, wa, wu, b)

  sub = xs
  for l in range(3):
    p = p_ss0 if l == 0 else sc_ss(sub, src_ss, dst_ss, zrows)
    wr = (params[f"Wr_ss_{l}"] + params[f"Wr_as_{l}"] + params[f"Wr_us_{l}"])
    sub = _tc_layer(l > 0, p, p_cnt, sub, params[f"Wl_ss_{l}"], wr, c_all[l])

  wf = jnp.zeros((D, D), f32).at[:, :OUT].set(params["Wf"])
  bf = jnp.full((1, D), -1e30, f32).at[0, :OUT].set(params["bf"])
  probs = _tc_final(sub, wf, bf)
  return probs[:N, :OUT]


# speedup vs baseline: 8.6587x; 1.2947x over previous
import functools

import jax
import jax.numpy as jnp
from jax import lax
from jax.experimental import pallas as pl
from jax.experimental.pallas import tpu as pltpu
from jax.experimental.pallas import tpu_sc as plsc

N = 10000
NP = 10240
D = 128
OUT = 16
NC = 2
NS = 16
NW = NC * NS
CH = 125
K = 2
E_SS_ = 320000
E_AS_ = 400000
STRIPE = NP // NS

def _seg_accumulate(table, src3, dst3, wid, srcv, dstv, rows, acc, nchunks,
                    isems, gsems, ssems, ones=None, cntacc=None, csem=None):
  nbodies = nchunks // (2 * K)

  def stage(r, q, sem):
    pltpu.async_copy(src3.at[wid, pl.ds(r * K, K)], srcv.at[q], sem)
    pltpu.async_copy(dst3.at[wid, pl.ds(r * K, K)], dstv.at[q], sem)

  def stage_wait(r, q, sem):
    pltpu.make_async_copy(
        src3.at[wid, pl.ds(r * K, K)], srcv.at[q], sem).wait()
    pltpu.make_async_copy(
        dst3.at[wid, pl.ds(r * K, K)], dstv.at[q], sem).wait()

  stage(0, 0, isems[0])

  def body(i, _):
    for q in (0, 1):
      r = 2 * i + q
      stage_wait(r, q, isems[q])
      if cntacc is not None:
        def drain_cnt():
          for b in range(K):
            pltpu.make_async_copy(
                ones, cntacc.at[dstv.at[1 - q, b]], csem).wait()
        if q == 0:
          pl.when(i > 0)(drain_cnt)
        else:
          drain_cnt()
      for b in range(K):
        def wait_sc(b=b):
          pltpu.make_async_copy(
              rows.at[b], acc.at[dstv.at[1 - q, b]], ssems[b]).wait()
        if q == 0:
          pl.when(i > 0)(wait_sc)
        else:
          wait_sc()
        pltpu.async_copy(table.at[srcv.at[q, b]], rows.at[b], gsems[b])
      if q == 0:
        stage(r + 1, 1, isems[1])
      else:
        def prefetch():
          stage(r + 1, 0, isems[0])
        pl.when(r + 1 < nchunks // K)(prefetch)
      for b in range(K):
        pltpu.make_async_copy(
            table.at[srcv.at[q, b]], rows.at[b], gsems[b]).wait()
        pltpu.async_copy(rows.at[b], acc.at[dstv.at[q, b]], ssems[b],
                         add=True)
        if cntacc is not None:
          pltpu.async_copy(ones, cntacc.at[dstv.at[q, b]], csem, add=True)
    return ()

  lax.fori_loop(0, nbodies, body, (), unroll=False)
  for b in range(K):
    pltpu.make_async_copy(rows.at[b], acc.at[dstv.at[1, b]], ssems[b]).wait()
    if cntacc is not None:
      pltpu.make_async_copy(ones, cntacc.at[dstv.at[1, b]], csem).wait()


def _zero_acc(zrows, acc, s):
  pltpu.sync_copy(zrows, acc.at[pl.ds(s * STRIPE, STRIPE)])


def _copy_out(acc, out, c, s):
  pltpu.sync_copy(acc.at[pl.ds(s * STRIPE, STRIPE)],
                  out.at[c, pl.ds(s * STRIPE, STRIPE)])


def _sc_multi_body(xs, xa, xu, src_as, dst_as, src_us, dst_us, src_ss, dst_ss,
                   zrows, zvec, ones_h,
                   p_as, p_us, p_ss, p_cnt,
                   acc, cntacc, srcv, dstv, rows, ones, *sems):
  c = lax.axis_index("c")
  s = lax.axis_index("s")
  wid = c * NS + s
  n_asus = E_AS_ // NW // CH
  n_ss = E_SS_ // NW // CH
  isems = sems[0:2]
  gsems = sems[2:2 + K]
  ssems = sems[2 + K:2 + 2 * K]
  csem = sems[2 + 2 * K]

  pltpu.sync_copy(ones_h, ones)

  _zero_acc(zrows, acc, s)
  plsc.subcore_barrier()
  _seg_accumulate(xa, src_as, dst_as, wid, srcv, dstv, rows, acc, n_asus,
                  isems, gsems, ssems)
  plsc.subcore_barrier()
  _copy_out(acc, p_as, c, s)
  plsc.subcore_barrier()

  _zero_acc(zrows, acc, s)
  plsc.subcore_barrier()
  _seg_accumulate(xu, src_us, dst_us, wid, srcv, dstv, rows, acc, n_asus,
                  isems, gsems, ssems)
  plsc.subcore_barrier()
  _copy_out(acc, p_us, c, s)
  plsc.subcore_barrier()

  _zero_acc(zrows, acc, s)
  pltpu.sync_copy(zvec, cntacc.at[pl.ds(s * STRIPE, STRIPE)])
  plsc.subcore_barrier()
  _seg_accumulate(xs, src_ss, dst_ss, wid, srcv, dstv, rows, acc, n_ss,
                  isems, gsems, ssems, ones=ones, cntacc=cntacc, csem=csem)
  plsc.subcore_barrier()
  _copy_out(acc, p_ss, c, s)
  pltpu.sync_copy(cntacc.at[pl.ds(s * STRIPE, STRIPE)],
                  p_cnt.at[c, pl.ds(s * STRIPE, STRIPE)])


def _sc_ss_body(xs, src_ss, dst_ss, zrows, p_ss, acc, srcv, dstv, rows, *sems):
  c = lax.axis_index("c")
  s = lax.axis_index("s")
  wid = c * NS + s
  n_ss = E_SS_ // NW // CH
  isems = sems[0:2]
  gsems = sems[2:2 + K]
  ssems = sems[2 + K:2 + 2 * K]
  _zero_acc(zrows, acc, s)
  plsc.subcore_barrier()
  _seg_accumulate(xs, src_ss, dst_ss, wid, srcv, dstv, rows, acc, n_ss,
                  isems, gsems, ssems)
  plsc.subcore_barrier()
  _copy_out(acc, p_ss, c, s)


@functools.cache
def _sc_kernels():
  mesh = plsc.VectorSubcoreMesh(
      core_axis_name="c", subcore_axis_name="s",
      num_cores=NC, num_subcores=NS)
  f32 = jnp.float32
  sc_multi = pl.kernel(
      _sc_multi_body,
      out_type=(
          jax.ShapeDtypeStruct((NC, NP, D), f32),
          jax.ShapeDtypeStruct((NC, NP, D), f32),
          jax.ShapeDtypeStruct((NC, NP, D), f32),
          jax.ShapeDtypeStruct((NC, NP), f32),
      ),
      mesh=mesh,
      scratch_types=[
          pltpu.VMEM_SHARED((NP, D), f32),
          pltpu.VMEM_SHARED((NP,), f32),
          pltpu.VMEM((2, K, CH), jnp.int32),
          pltpu.VMEM((2, K, CH), jnp.int32),
          pltpu.VMEM((K, CH, D), f32),
          pltpu.VMEM((CH,), f32),
      ] + [pltpu.SemaphoreType.DMA] * (3 + 2 * K),
  )
  sc_ss = pl.kernel(
      _sc_ss_body,
      out_type=jax.ShapeDtypeStruct((NC, NP, D), f32),
      mesh=mesh,
      scratch_types=[
          pltpu.VMEM_SHARED((NP, D), f32),
          pltpu.VMEM((2, K, CH), jnp.int32),
          pltpu.VMEM((2, K, CH), jnp.int32),
          pltpu.VMEM((K, CH, D), f32),
      ] + [pltpu.SemaphoreType.DMA] * (2 + 2 * K),
  )
  return sc_multi, sc_ss


BR = 1024


def _tc_prep_body(pas_ref, pus_ref, wa_ref, wu_ref, b_ref, out_ref):
  a = pas_ref[0] + pas_ref[1]
  u = pus_ref[0] + pus_ref[1]
  o = jnp.dot(a, wa_ref[0], preferred_element_type=jnp.float32)
  o = o + jnp.dot(u, wu_ref[0], preferred_element_type=jnp.float32)
  out_ref[0] = o + b_ref[0, 0][None, :]


def _tc_prep(p_as, p_us, wa, wu, b):
  return pl.pallas_call(
      _tc_prep_body,
      out_shape=jax.ShapeDtypeStruct((3, NP, D), jnp.float32),
      grid=(3, NP // BR),
      in_specs=[
          pl.BlockSpec((NC, BR, D), lambda l, i: (0, i, 0)),
          pl.BlockSpec((NC, BR, D), lambda l, i: (0, i, 0)),
          pl.BlockSpec((1, D, D), lambda l, i: (l, 0, 0)),
          pl.BlockSpec((1, D, D), lambda l, i: (l, 0, 0)),
          pl.BlockSpec((1, 1, D), lambda l, i: (l, 0, 0)),
      ],
      out_specs=pl.BlockSpec((1, BR, D), lambda l, i: (l, i, 0)),
  )(p_as, p_us, wa, wu, b)


def _tc_layer_body(mean, p_ref, cnt_ref, sub_ref, wl_ref, wr_ref, c_ref,
                   out_ref):
  y = p_ref[0] + p_ref[1]
  if mean:
    cnt = cnt_ref[0] + cnt_ref[1]
    y = y * (1.0 / jnp.maximum(cnt, 1.0))[:, None]
  o = jnp.dot(y, wl_ref[...], preferred_element_type=jnp.float32)
  o = o + jnp.dot(sub_ref[...], wr_ref[...], preferred_element_type=jnp.float32)
  out_ref[...] = jnp.maximum(o + c_ref[...], 0.0)


def _tc_layer(mean, p, cnt, sub, wl, wr, cterm):
  return pl.pallas_call(
      functools.partial(_tc_layer_body, mean),
      out_shape=jax.ShapeDtypeStruct((NP, D), jnp.float32),
      grid=(NP // BR,),
      in_specs=[
          pl.BlockSpec((NC, BR, D), lambda i: (0, i, 0)),
          pl.BlockSpec((NC, BR), lambda i: (0, i)),
          pl.BlockSpec((BR, D), lambda i: (i, 0)),
          pl.BlockSpec((D, D), lambda i: (0, 0)),
          pl.BlockSpec((D, D), lambda i: (0, 0)),
          pl.BlockSpec((BR, D), lambda i: (i, 0)),
      ],
      out_specs=pl.BlockSpec((BR, D), lambda i: (i, 0)),
  )(p, cnt, sub, wl, wr, cterm)


def _tc_final_body(sub_ref, wf_ref, bf_ref, out_ref):
  logits = jnp.dot(sub_ref[...], wf_ref[...],
                   preferred_element_type=jnp.float32) + bf_ref[0][None, :]
  m = jnp.max(logits, axis=1, keepdims=True)
  e = jnp.exp(logits - m)
  out_ref[...] = e / jnp.sum(e, axis=1, keepdims=True)


def _tc_final(sub, wf, bf):
  return pl.pallas_call(
      _tc_final_body,
      out_shape=jax.ShapeDtypeStruct((NP, D), jnp.float32),
      grid=(NP // BR,),
      in_specs=[
          pl.BlockSpec((BR, D), lambda i: (i, 0)),
          pl.BlockSpec((D, D), lambda i: (0, 0)),
          pl.BlockSpec((1, D), lambda i: (0, 0)),
      ],
      out_specs=pl.BlockSpec((BR, D), lambda i: (i, 0)),
  )(sub, wf, bf)


def kernel(x_sub, x_hru_agr, x_hru_urb, ei_ss, ei_as, ei_us, ei_sa, ei_su,
           params):
  del ei_sa, ei_su
  f32 = jnp.float32
  xs = jnp.zeros((NP, D), f32).at[:N].set(x_sub.astype(f32))

  def reshape_ei(ei):
    src = ei[0].astype(jnp.int32).reshape(NW, -1, CH)
    dst = ei[1].astype(jnp.int32).reshape(NW, -1, CH)
    return src, dst

  src_ss, dst_ss = reshape_ei(ei_ss)
  src_as, dst_as = reshape_ei(ei_as)
  src_us, dst_us = reshape_ei(ei_us)

  zrows = jnp.zeros((STRIPE, D), f32)
  zvec = jnp.zeros((STRIPE,), f32)
  ones_h = jnp.ones((CH,), f32)

  sc_multi, sc_ss = _sc_kernels()
  p_as, p_us, p_ss0, p_cnt = sc_multi(
      xs, x_hru_agr.astype(f32), x_hru_urb.astype(f32),
      src_as, dst_as, src_us, dst_us, src_ss, dst_ss, zrows, zvec, ones_h)

  wa = jnp.stack([params[f"Wl_as_{l}"] for l in range(3)])
  wu = jnp.stack([params[f"Wl_us_{l}"] for l in range(3)])
  b = jnp.stack([params[f"bl_ss_{l}"] + params[f"bl_as_{l}"]
                 + params[f"bl_us_{l}"] for l in range(3)])[:, None, :]
  c_all = _tc_prep(p_as, p_us, wa, wu, b)

  sub = xs
  for l in range(3):
    p = p_ss0 if l == 0 else sc_ss(sub, src_ss, dst_ss, zrows)
    wr = (params[f"Wr_ss_{l}"] + params[f"Wr_as_{l}"] + params[f"Wr_us_{l}"])
    sub = _tc_layer(l > 0, p, p_cnt, sub, params[f"Wl_ss_{l}"], wr, c_all[l])

  wf = jnp.zeros((D, D), f32).at[:, :OUT].set(params["Wf"])
  bf = jnp.full((1, D), -1e30, f32).at[0, :OUT].set(params["bf"])
  probs = _tc_final(sub, wf, bf)
  return probs[:N, :OUT]


# speedup vs baseline: 9.8608x; 1.1388x over previous
import functools

import jax
import jax.numpy as jnp
from jax import lax
from jax.experimental import pallas as pl
from jax.experimental.pallas import tpu as pltpu
from jax.experimental.pallas import tpu_sc as plsc

N = 10000
NP = 10240
D = 128
OUT = 16
NC = 2
NS = 16
NW = NC * NS
CH = 50
K = 5
E_SS_ = 320000
E_AS_ = 400000
STRIPE = NP // NS

def _cnt_pass(dst3, wid, dstv, ones, cntacc, nchunks, csem):
  g2 = 2 * K

  def body(i, _):
    def drain():
      for q in (0, 1):
        for b in range(K):
          pltpu.make_async_copy(ones, cntacc.at[dstv.at[q, b]], csem).wait()
    pl.when(i > 0)(drain)
    base = wid * (nchunks // K)
    pltpu.sync_copy(dst3.at[base + 2 * i], dstv.at[0])
    pltpu.sync_copy(dst3.at[base + 2 * i + 1], dstv.at[1])
    for q in (0, 1):
      for b in range(K):
        pltpu.async_copy(ones, cntacc.at[dstv.at[q, b]], csem, add=True)
    return ()

  lax.fori_loop(0, nchunks // g2, body, (), unroll=False)
  for q in (0, 1):
    for b in range(K):
      pltpu.make_async_copy(ones, cntacc.at[dstv.at[q, b]], csem).wait()


def _seg_accumulate(table, src3, dst3, wid, srcv, dstv, rows, acc, nchunks,
                    isems, gsems, ssems):
  nbodies = nchunks // (2 * K)

  base = wid * (nchunks // K)

  def stage(r, q, sem):
    pltpu.async_copy(src3.at[base + r], srcv.at[q], sem)
    pltpu.async_copy(dst3.at[base + r], dstv.at[q], sem)

  def stage_wait(r, q, sem):
    pltpu.make_async_copy(src3.at[base + r], srcv.at[q], sem).wait()
    pltpu.make_async_copy(dst3.at[base + r], dstv.at[q], sem).wait()

  stage(0, 0, isems[0])

  def body(i, _):
    for q in (0, 1):
      r = 2 * i + q
      stage_wait(r, q, isems[q])
      for b in range(K):
        def wait_sc(b=b):
          pltpu.make_async_copy(
              rows.at[b], acc.at[dstv.at[1 - q, b]], ssems[b]).wait()
        if q == 0:
          pl.when(i > 0)(wait_sc)
        else:
          wait_sc()
        pltpu.async_copy(table.at[srcv.at[q, b]], rows.at[b], gsems[b])
      if q == 0:
        stage(r + 1, 1, isems[1])
      else:
        def prefetch():
          stage(r + 1, 0, isems[0])
        pl.when(r + 1 < nchunks // K)(prefetch)
      for b in range(K):
        pltpu.make_async_copy(
            table.at[srcv.at[q, b]], rows.at[b], gsems[b]).wait()
        pltpu.async_copy(rows.at[b], acc.at[dstv.at[q, b]], ssems[b],
                         add=True)
    return ()

  lax.fori_loop(0, nbodies, body, (), unroll=False)
  for b in range(K):
    pltpu.make_async_copy(rows.at[b], acc.at[dstv.at[1, b]], ssems[b]).wait()


def _zero_acc(zrows, acc, s):
  pltpu.sync_copy(zrows, acc.at[pl.ds(s * STRIPE, STRIPE)])


def _copy_out(acc, out, c, s):
  pltpu.sync_copy(acc.at[pl.ds(s * STRIPE, STRIPE)],
                  out.at[c, pl.ds(s * STRIPE, STRIPE)])


def _sc_multi_body(xs, xa, xu, src_as, dst_as, src_us, dst_us, src_ss, dst_ss,
                   zrows, zvec, ones_h,
                   p_as, p_us, p_ss, p_cnt,
                   acc, cntacc, srcv, dstv, rows, ones, *sems):
  c = lax.axis_index("c")
  s = lax.axis_index("s")
  wid = c * NS + s
  n_asus = E_AS_ // NW // CH
  n_ss = E_SS_ // NW // CH
  isems = sems[0:2]
  gsems = sems[2:2 + K]
  ssems = sems[2 + K:2 + 2 * K]
  csem = sems[2 + 2 * K]

  pltpu.sync_copy(ones_h, ones)

  _zero_acc(zrows, acc, s)
  plsc.subcore_barrier()
  _seg_accumulate(xa, src_as, dst_as, wid, srcv, dstv, rows, acc, n_asus,
                  isems, gsems, ssems)
  plsc.subcore_barrier()
  _copy_out(acc, p_as, c, s)
  plsc.subcore_barrier()

  _zero_acc(zrows, acc, s)
  plsc.subcore_barrier()
  _seg_accumulate(xu, src_us, dst_us, wid, srcv, dstv, rows, acc, n_asus,
                  isems, gsems, ssems)
  plsc.subcore_barrier()
  _copy_out(acc, p_us, c, s)
  plsc.subcore_barrier()

  _zero_acc(zrows, acc, s)
  pltpu.sync_copy(zvec, cntacc.at[pl.ds(s * STRIPE, STRIPE)])
  plsc.subcore_barrier()
  _seg_accumulate(xs, src_ss, dst_ss, wid, srcv, dstv, rows, acc, n_ss,
                  isems, gsems, ssems)
  _cnt_pass(dst_ss, wid, dstv, ones, cntacc, n_ss, csem)
  plsc.subcore_barrier()
  _copy_out(acc, p_ss, c, s)
  pltpu.sync_copy(cntacc.at[pl.ds(s * STRIPE, STRIPE)],
                  p_cnt.at[c, pl.ds(s * STRIPE, STRIPE)])


def _sc_ss_body(xs, src_ss, dst_ss, zrows, p_ss, acc, srcv, dstv, rows, *sems):
  c = lax.axis_index("c")
  s = lax.axis_index("s")
  wid = c * NS + s
  n_ss = E_SS_ // NW // CH
  isems = sems[0:2]
  gsems = sems[2:2 + K]
  ssems = sems[2 + K:2 + 2 * K]
  _zero_acc(zrows, acc, s)
  plsc.subcore_barrier()
  _seg_accumulate(xs, src_ss, dst_ss, wid, srcv, dstv, rows, acc, n_ss,
                  isems, gsems, ssems)
  plsc.subcore_barrier()
  _copy_out(acc, p_ss, c, s)


@functools.cache
def _sc_kernels():
  mesh = plsc.VectorSubcoreMesh(
      core_axis_name="c", subcore_axis_name="s",
      num_cores=NC, num_subcores=NS)
  f32 = jnp.float32
  sc_multi = pl.kernel(
      _sc_multi_body,
      out_type=(
          jax.ShapeDtypeStruct((NC, NP, D), f32),
          jax.ShapeDtypeStruct((NC, NP, D), f32),
          jax.ShapeDtypeStruct((NC, NP, D), f32),
          jax.ShapeDtypeStruct((NC, NP), f32),
      ),
      mesh=mesh,
      scratch_types=[
          pltpu.VMEM_SHARED((NP, D), f32),
          pltpu.VMEM_SHARED((NP,), f32),
          pltpu.VMEM((2, K, CH), jnp.int32),
          pltpu.VMEM((2, K, CH), jnp.int32),
          pltpu.VMEM((K, CH, D), f32),
          pltpu.VMEM((CH,), f32),
      ] + [pltpu.SemaphoreType.DMA] * (3 + 2 * K),
  )
  sc_ss = pl.kernel(
      _sc_ss_body,
      out_type=jax.ShapeDtypeStruct((NC, NP, D), f32),
      mesh=mesh,
      scratch_types=[
          pltpu.VMEM_SHARED((NP, D), f32),
          pltpu.VMEM((2, K, CH), jnp.int32),
          pltpu.VMEM((2, K, CH), jnp.int32),
          pltpu.VMEM((K, CH, D), f32),
      ] + [pltpu.SemaphoreType.DMA] * (2 + 2 * K),
  )
  return sc_multi, sc_ss


BR = 1024


def _tc_prep_body(pas_ref, pus_ref, wa_ref, wu_ref, b_ref, out_ref):
  a = pas_ref[0] + pas_ref[1]
  u = pus_ref[0] + pus_ref[1]
  o = jnp.dot(a, wa_ref[0], preferred_element_type=jnp.float32)
  o = o + jnp.dot(u, wu_ref[0], preferred_element_type=jnp.float32)
  out_ref[0] = o + b_ref[0, 0][None, :]


def _tc_prep(p_as, p_us, wa, wu, b):
  return pl.pallas_call(
      _tc_prep_body,
      out_shape=jax.ShapeDtypeStruct((3, NP, D), jnp.float32),
      grid=(3, NP // BR),
      in_specs=[
          pl.BlockSpec((NC, BR, D), lambda l, i: (0, i, 0)),
          pl.BlockSpec((NC, BR, D), lambda l, i: (0, i, 0)),
          pl.BlockSpec((1, D, D), lambda l, i: (l, 0, 0)),
          pl.BlockSpec((1, D, D), lambda l, i: (l, 0, 0)),
          pl.BlockSpec((1, 1, D), lambda l, i: (l, 0, 0)),
      ],
      out_specs=pl.BlockSpec((1, BR, D), lambda l, i: (l, i, 0)),
  )(p_as, p_us, wa, wu, b)


def _tc_layer_body(mean, p_ref, cnt_ref, sub_ref, wl_ref, wr_ref, c_ref,
                   out_ref):
  y = p_ref[0] + p_ref[1]
  if mean:
    cnt = cnt_ref[0] + cnt_ref[1]
    y = y * (1.0 / jnp.maximum(cnt, 1.0))[:, None]
  o = jnp.dot(y, wl_ref[...], preferred_element_type=jnp.float32)
  o = o + jnp.dot(sub_ref[...], wr_ref[...], preferred_element_type=jnp.float32)
  out_ref[...] = jnp.maximum(o + c_ref[...], 0.0)


def _tc_layer(mean, p, cnt, sub, wl, wr, cterm):
  return pl.pallas_call(
      functools.partial(_tc_layer_body, mean),
      out_shape=jax.ShapeDtypeStruct((NP, D), jnp.float32),
      grid=(NP // BR,),
      in_specs=[
          pl.BlockSpec((NC, BR, D), lambda i: (0, i, 0)),
          pl.BlockSpec((NC, BR), lambda i: (0, i)),
          pl.BlockSpec((BR, D), lambda i: (i, 0)),
          pl.BlockSpec((D, D), lambda i: (0, 0)),
          pl.BlockSpec((D, D), lambda i: (0, 0)),
          pl.BlockSpec((BR, D), lambda i: (i, 0)),
      ],
      out_specs=pl.BlockSpec((BR, D), lambda i: (i, 0)),
  )(p, cnt, sub, wl, wr, cterm)


def _tc_final_body(sub_ref, wf_ref, bf_ref, out_ref):
  logits = jnp.dot(sub_ref[...], wf_ref[...],
                   preferred_element_type=jnp.float32) + bf_ref[0][None, :]
  m = jnp.max(logits, axis=1, keepdims=True)
  e = jnp.exp(logits - m)
  out_ref[...] = e / jnp.sum(e, axis=1, keepdims=True)


def _tc_final(sub, wf, bf):
  return pl.pallas_call(
      _tc_final_body,
      out_shape=jax.ShapeDtypeStruct((NP, D), jnp.float32),
      grid=(NP // BR,),
      in_specs=[
          pl.BlockSpec((BR, D), lambda i: (i, 0)),
          pl.BlockSpec((D, D), lambda i: (0, 0)),
          pl.BlockSpec((1, D), lambda i: (0, 0)),
      ],
      out_specs=pl.BlockSpec((BR, D), lambda i: (i, 0)),
  )(sub, wf, bf)


def kernel(x_sub, x_hru_agr, x_hru_urb, ei_ss, ei_as, ei_us, ei_sa, ei_su,
           params):
  del ei_sa, ei_su
  f32 = jnp.float32
  xs = jnp.zeros((NP, D), f32).at[:N].set(x_sub.astype(f32))

  def reshape_ei(ei):
    src = ei[0].astype(jnp.int32).reshape(-1, K, CH)
    dst = ei[1].astype(jnp.int32).reshape(-1, K, CH)
    return src, dst

  src_ss, dst_ss = reshape_ei(ei_ss)
  src_as, dst_as = reshape_ei(ei_as)
  src_us, dst_us = reshape_ei(ei_us)

  zrows = jnp.zeros((STRIPE, D), f32)
  zvec = jnp.zeros((STRIPE,), f32)
  ones_h = jnp.ones((CH,), f32)

  sc_multi, sc_ss = _sc_kernels()
  p_as, p_us, p_ss0, p_cnt = sc_multi(
      xs, x_hru_agr.astype(f32), x_hru_urb.astype(f32),
      src_as, dst_as, src_us, dst_us, src_ss, dst_ss, zrows, zvec, ones_h)

  wa = jnp.stack([params[f"Wl_as_{l}"] for l in range(3)])
  wu = jnp.stack([params[f"Wl_us_{l}"] for l in range(3)])
  b = jnp.stack([params[f"bl_ss_{l}"] + params[f"bl_as_{l}"]
                 + params[f"bl_us_{l}"] for l in range(3)])[:, None, :]
  c_all = _tc_prep(p_as, p_us, wa, wu, b)

  sub = xs
  for l in range(3):
    p = p_ss0 if l == 0 else sc_ss(sub, src_ss, dst_ss, zrows)
    wr = (params[f"Wr_ss_{l}"] + params[f"Wr_as_{l}"] + params[f"Wr_us_{l}"])
    sub = _tc_layer(l > 0, p, p_cnt, sub, params[f"Wl_ss_{l}"], wr, c_all[l])

  wf = jnp.zeros((D, D), f32).at[:, :OUT].set(params["Wf"])
  bf = jnp.full((1, D), -1e30, f32).at[0, :OUT].set(params["bf"])
  probs = _tc_final(sub, wf, bf)
  return probs[:N, :OUT]


# speedup vs baseline: 10.4529x; 1.0601x over previous
import functools

import jax
import jax.numpy as jnp
from jax import lax
from jax.experimental import pallas as pl
from jax.experimental.pallas import tpu as pltpu
from jax.experimental.pallas import tpu_sc as plsc

N = 10000
NP = 10240
D = 128
OUT = 16
NC = 2
NS = 16
NW = NC * NS
CH = 50
K = 5
E_SS_ = 320000
E_AS_ = 400000
STRIPE = NP // NS

def _cnt_pass(dst3, wid, dstv, ones, cntacc, nchunks, csem):
  g2 = 2 * K

  def body(i, _):
    def drain():
      for q in (0, 1):
        for b in range(K):
          pltpu.make_async_copy(ones, cntacc.at[dstv.at[q, b]], csem).wait()
    pl.when(i > 0)(drain)
    base = wid * (nchunks // K)
    pltpu.sync_copy(dst3.at[base + 2 * i], dstv.at[0])
    pltpu.sync_copy(dst3.at[base + 2 * i + 1], dstv.at[1])
    for q in (0, 1):
      for b in range(K):
        pltpu.async_copy(ones, cntacc.at[dstv.at[q, b]], csem, add=True)
    return ()

  lax.fori_loop(0, nchunks // g2, body, (), unroll=False)
  for q in (0, 1):
    for b in range(K):
      pltpu.make_async_copy(ones, cntacc.at[dstv.at[q, b]], csem).wait()


def _seg_accumulate(table, src3, dst3, wid, srcv, dstv, rows, acc, nchunks,
                    isems, gsems, ssems):
  nbodies = nchunks // (2 * K)

  base = wid * (nchunks // K)

  def stage(r, q, sem):
    pltpu.async_copy(src3.at[base + r], srcv.at[q], sem)
    pltpu.async_copy(dst3.at[base + r], dstv.at[q], sem)

  def stage_wait(r, q, sem):
    pltpu.make_async_copy(src3.at[base + r], srcv.at[q], sem).wait()
    pltpu.make_async_copy(dst3.at[base + r], dstv.at[q], sem).wait()

  stage(0, 0, isems[0])

  def body(i, _):
    for q in (0, 1):
      r = 2 * i + q
      stage_wait(r, q, isems[q])
      for b in range(K):
        def wait_sc(b=b):
          pltpu.make_async_copy(
              rows.at[b], acc.at[dstv.at[1 - q, b]], ssems[b]).wait()
        if q == 0:
          pl.when(i > 0)(wait_sc)
        else:
          wait_sc()
        pltpu.async_copy(table.at[srcv.at[q, b]], rows.at[b], gsems[b])
      if q == 0:
        stage(r + 1, 1, isems[1])
      else:
        def prefetch():
          stage(r + 1, 0, isems[0])
        pl.when(r + 1 < nchunks // K)(prefetch)
      for b in range(K):
        pltpu.make_async_copy(
            table.at[srcv.at[q, b]], rows.at[b], gsems[b]).wait()
        pltpu.async_copy(rows.at[b], acc.at[dstv.at[q, b]], ssems[b],
                         add=True)
    return ()

  lax.fori_loop(0, nbodies, body, (), unroll=False)
  for b in range(K):
    pltpu.make_async_copy(rows.at[b], acc.at[dstv.at[1, b]], ssems[b]).wait()


def _zero_acc(zrows, acc, s):
  pltpu.sync_copy(zrows, acc.at[pl.ds(s * STRIPE, STRIPE)])


def _copy_out(acc, out, c, s):
  pltpu.sync_copy(acc.at[pl.ds(s * STRIPE, STRIPE)],
                  out.at[c, pl.ds(s * STRIPE, STRIPE)])


def _sc_multi_body(xs, xa, xu, src_as, dst_as, src_us, dst_us, src_ss, dst_ss,
                   zrows, zvec, ones_h,
                   p_as, p_us, p_ss, p_cnt,
                   acc, cntacc, srcv, dstv, rows, ones, *sems):
  c = lax.axis_index("c")
  s = lax.axis_index("s")
  wid = c * NS + s
  n_asus = E_AS_ // NW // CH
  n_ss = E_SS_ // NW // CH
  isems = sems[0:2]
  gsems = sems[2:2 + K]
  ssems = sems[2 + K:2 + 2 * K]
  csem = sems[2 + 2 * K]

  pltpu.sync_copy(ones_h, ones)

  _zero_acc(zrows, acc, s)
  plsc.subcore_barrier()
  _seg_accumulate(xa, src_as, dst_as, wid, srcv, dstv, rows, acc, n_asus,
                  isems, gsems, ssems)
  plsc.subcore_barrier()
  _copy_out(acc, p_as, c, s)
  plsc.subcore_barrier()

  _zero_acc(zrows, acc, s)
  plsc.subcore_barrier()
  _seg_accumulate(xu, src_us, dst_us, wid, srcv, dstv, rows, acc, n_asus,
                  isems, gsems, ssems)
  plsc.subcore_barrier()
  _copy_out(acc, p_us, c, s)
  plsc.subcore_barrier()

  _zero_acc(zrows, acc, s)
  pltpu.sync_copy(zvec, cntacc.at[pl.ds(s * STRIPE, STRIPE)])
  plsc.subcore_barrier()
  _seg_accumulate(xs, src_ss, dst_ss, wid, srcv, dstv, rows, acc, n_ss,
                  isems, gsems, ssems)
  _cnt_pass(dst_ss, wid, dstv, ones, cntacc, n_ss, csem)
  plsc.subcore_barrier()
  _copy_out(acc, p_ss, c, s)
  pltpu.sync_copy(cntacc.at[pl.ds(s * STRIPE, STRIPE)],
                  p_cnt.at[c, pl.ds(s * STRIPE, STRIPE)])


def _sc_ss_body(xs, src_ss, dst_ss, zrows, p_ss, acc, srcv, dstv, rows, *sems):
  c = lax.axis_index("c")
  s = lax.axis_index("s")
  wid = c * NS + s
  n_ss = E_SS_ // NW // CH
  isems = sems[0:2]
  gsems = sems[2:2 + K]
  ssems = sems[2 + K:2 + 2 * K]
  _zero_acc(zrows, acc, s)
  plsc.subcore_barrier()
  _seg_accumulate(xs, src_ss, dst_ss, wid, srcv, dstv, rows, acc, n_ss,
                  isems, gsems, ssems)
  plsc.subcore_barrier()
  _copy_out(acc, p_ss, c, s)


@functools.cache
def _sc_kernels():
  mesh = plsc.VectorSubcoreMesh(
      core_axis_name="c", subcore_axis_name="s",
      num_cores=NC, num_subcores=NS)
  f32 = jnp.float32
  sc_multi = pl.kernel(
      _sc_multi_body,
      out_type=(
          jax.ShapeDtypeStruct((NC, NP, D), f32),
          jax.ShapeDtypeStruct((NC, NP, D), f32),
          jax.ShapeDtypeStruct((NC, NP, D), f32),
          jax.ShapeDtypeStruct((NC, NP), f32),
      ),
      mesh=mesh,
      scratch_types=[
          pltpu.VMEM_SHARED((NP, D), f32),
          pltpu.VMEM_SHARED((NP,), f32),
          pltpu.VMEM((2, K, CH), jnp.int32),
          pltpu.VMEM((2, K, CH), jnp.int32),
          pltpu.VMEM((K, CH, D), f32),
          pltpu.VMEM((CH,), f32),
      ] + [pltpu.SemaphoreType.DMA] * (3 + 2 * K),
  )
  sc_ss = pl.kernel(
      _sc_ss_body,
      out_type=jax.ShapeDtypeStruct((NC, NP, D), f32),
      mesh=mesh,
      scratch_types=[
          pltpu.VMEM_SHARED((NP, D), f32),
          pltpu.VMEM((2, K, CH), jnp.int32),
          pltpu.VMEM((2, K, CH), jnp.int32),
          pltpu.VMEM((K, CH, D), f32),
      ] + [pltpu.SemaphoreType.DMA] * (2 + 2 * K),
  )
  return sc_multi, sc_ss


BR = 1024


def _dot(a, b):
  return jnp.dot(a, b, preferred_element_type=jnp.float32)


def _tc_layer0_body(pss_ref, pas_ref, pus_ref, sub_ref, wl_ref, wr_ref,
                    wa_ref, wu_ref, b_ref, out_ref, c1_ref, c2_ref):
  a = pas_ref[0] + pas_ref[1]
  u = pus_ref[0] + pus_ref[1]
  c0 = _dot(a, wa_ref[0]) + _dot(u, wu_ref[0]) + b_ref[0, 0][None, :]
  y = pss_ref[0] + pss_ref[1]
  o = _dot(y, wl_ref[...]) + _dot(sub_ref[...], wr_ref[...]) + c0
  out_ref[...] = jnp.maximum(o, 0.0)
  c1_ref[...] = _dot(a, wa_ref[1]) + _dot(u, wu_ref[1]) + b_ref[1, 0][None, :]
  c2_ref[...] = _dot(a, wa_ref[2]) + _dot(u, wu_ref[2]) + b_ref[2, 0][None, :]


def _tc_layer0(p_ss, p_as, p_us, sub, wl, wr, wa, wu, b):
  f32 = jnp.float32
  return pl.pallas_call(
      _tc_layer0_body,
      out_shape=(jax.ShapeDtypeStruct((NP, D), f32),
                 jax.ShapeDtypeStruct((NP, D), f32),
                 jax.ShapeDtypeStruct((NP, D), f32)),
      grid=(NP // BR,),
      in_specs=[
          pl.BlockSpec((NC, BR, D), lambda i: (0, i, 0)),
          pl.BlockSpec((NC, BR, D), lambda i: (0, i, 0)),
          pl.BlockSpec((NC, BR, D), lambda i: (0, i, 0)),
          pl.BlockSpec((BR, D), lambda i: (i, 0)),
          pl.BlockSpec((D, D), lambda i: (0, 0)),
          pl.BlockSpec((D, D), lambda i: (0, 0)),
          pl.BlockSpec((3, D, D), lambda i: (0, 0, 0)),
          pl.BlockSpec((3, D, D), lambda i: (0, 0, 0)),
          pl.BlockSpec((3, 1, D), lambda i: (0, 0, 0)),
      ],
      out_specs=(pl.BlockSpec((BR, D), lambda i: (i, 0)),
                 pl.BlockSpec((BR, D), lambda i: (i, 0)),
                 pl.BlockSpec((BR, D), lambda i: (i, 0))),
  )(p_ss, p_as, p_us, sub, wl, wr, wa, wu, b)


def _tc_layer1_body(p_ref, cnt_ref, sub_ref, wl_ref, wr_ref, c_ref, out_ref):
  cnt = cnt_ref[0] + cnt_ref[1]
  y = (p_ref[0] + p_ref[1]) * (1.0 / jnp.maximum(cnt, 1.0))[:, None]
  o = _dot(y, wl_ref[...]) + _dot(sub_ref[...], wr_ref[...]) + c_ref[...]
  out_ref[...] = jnp.maximum(o, 0.0)


def _tc_layer1(p, cnt, sub, wl, wr, cterm):
  return pl.pallas_call(
      _tc_layer1_body,
      out_shape=jax.ShapeDtypeStruct((NP, D), jnp.float32),
      grid=(NP // BR,),
      in_specs=[
          pl.BlockSpec((NC, BR, D), lambda i: (0, i, 0)),
          pl.BlockSpec((NC, BR), lambda i: (0, i)),
          pl.BlockSpec((BR, D), lambda i: (i, 0)),
          pl.BlockSpec((D, D), lambda i: (0, 0)),
          pl.BlockSpec((D, D), lambda i: (0, 0)),
          pl.BlockSpec((BR, D), lambda i: (i, 0)),
      ],
      out_specs=pl.BlockSpec((BR, D), lambda i: (i, 0)),
  )(p, cnt, sub, wl, wr, cterm)


def _tc_layer2f_body(p_ref, cnt_ref, sub_ref, wl_ref, wr_ref, c_ref,
                     wf_ref, bf_ref, out_ref):
  cnt = cnt_ref[0] + cnt_ref[1]
  y = (p_ref[0] + p_ref[1]) * (1.0 / jnp.maximum(cnt, 1.0))[:, None]
  o = _dot(y, wl_ref[...]) + _dot(sub_ref[...], wr_ref[...]) + c_ref[...]
  sub3 = jnp.maximum(o, 0.0)
  logits = _dot(sub3, wf_ref[...]) + bf_ref[0][None, :]
  m = jnp.max(logits, axis=1, keepdims=True)
  e = jnp.exp(logits - m)
  out_ref[...] = e / jnp.sum(e, axis=1, keepdims=True)


def _tc_layer2f(p, cnt, sub, wl, wr, cterm, wf, bf):
  return pl.pallas_call(
      _tc_layer2f_body,
      out_shape=jax.ShapeDtypeStruct((NP, D), jnp.float32),
      grid=(NP // BR,),
      in_specs=[
          pl.BlockSpec((NC, BR, D), lambda i: (0, i, 0)),
          pl.BlockSpec((NC, BR), lambda i: (0, i)),
          pl.BlockSpec((BR, D), lambda i: (i, 0)),
          pl.BlockSpec((D, D), lambda i: (0, 0)),
          pl.BlockSpec((D, D), lambda i: (0, 0)),
          pl.BlockSpec((BR, D), lambda i: (i, 0)),
          pl.BlockSpec((D, D), lambda i: (0, 0)),
          pl.BlockSpec((1, D), lambda i: (0, 0)),
      ],
      out_specs=pl.BlockSpec((BR, D), lambda i: (i, 0)),
  )(p, cnt, sub, wl, wr, cterm, wf, bf)


def kernel(x_sub, x_hru_agr, x_hru_urb, ei_ss, ei_as, ei_us, ei_sa, ei_su,
           params):
  del ei_sa, ei_su
  f32 = jnp.float32
  xs = jnp.zeros((NP, D), f32).at[:N].set(x_sub.astype(f32))

  def reshape_ei(ei):
    src = ei[0].astype(jnp.int32).reshape(-1, K, CH)
    dst = ei[1].astype(jnp.int32).reshape(-1, K, CH)
    return src, dst

  src_ss, dst_ss = reshape_ei(ei_ss)
  src_as, dst_as = reshape_ei(ei_as)
  src_us, dst_us = reshape_ei(ei_us)

  zrows = jnp.zeros((STRIPE, D), f32)
  zvec = jnp.zeros((STRIPE,), f32)
  ones_h = jnp.ones((CH,), f32)

  sc_multi, sc_ss = _sc_kernels()
  p_as, p_us, p_ss0, p_cnt = sc_multi(
      xs, x_hru_agr.astype(f32), x_hru_urb.astype(f32),
      src_as, dst_as, src_us, dst_us, src_ss, dst_ss, zrows, zvec, ones_h)

  wa = jnp.stack([params[f"Wl_as_{l}"] for l in range(3)])
  wu = jnp.stack([params[f"Wl_us_{l}"] for l in range(3)])
  b = jnp.stack([params[f"bl_ss_{l}"] + params[f"bl_as_{l}"]
                 + params[f"bl_us_{l}"] for l in range(3)])[:, None, :]

  def wr_sum(l):
    return params[f"Wr_ss_{l}"] + params[f"Wr_as_{l}"] + params[f"Wr_us_{l}"]

  sub1, c1, c2 = _tc_layer0(p_ss0, p_as, p_us, xs, params["Wl_ss_0"],
                            wr_sum(0), wa, wu, b)
  p1 = sc_ss(sub1, src_ss, dst_ss, zrows)
  sub2 = _tc_layer1(p1, p_cnt, sub1, params["Wl_ss_1"], wr_sum(1), c1)
  p2 = sc_ss(sub2, src_ss, dst_ss, zrows)
  wf = jnp.zeros((D, D), f32).at[:, :OUT].set(params["Wf"])
  bf = jnp.full((1, D), -1e30, f32).at[0, :OUT].set(params["bf"])
  probs = _tc_layer2f(p2, p_cnt, sub2, params["Wl_ss_2"], wr_sum(2), c2,
                      wf, bf)
  return probs[:N, :OUT]
